# baseline (speedup 1.0000x reference)
import functools

import jax
import jax.numpy as jnp
from jax import lax
from jax.experimental import pallas as pl
from jax.experimental.pallas import tpu as pltpu
from jax.experimental.pallas import tpu_sc as plsc

N_CAT = 8
N_PAIR = 4
VOCAB = 1000
EMBED = 64
OUT_DIM = 512
LANE = 2 * EMBED

NC, NS = 2, 16
NW = NC * NS
CROWS = 64
GIDX = 2 * CROWS


def _gather_body(tab_hbm, idx_hbm, cat_hbm, idx_v, rows_v, sem_i, sem_g, sem_s):
    batch = cat_hbm.shape[1] // 2
    rows_per_w = batch // NW
    kchunks = rows_per_w // CROWS
    nunits = N_PAIR * kchunks
    wid = lax.axis_index("s") * NC + lax.axis_index("c")
    r0 = wid * rows_per_w

    for j in range(N_PAIR):
        h = pltpu.async_copy(
            idx_hbm.at[pl.ds(j * 2 * batch + 2 * r0, 2 * rows_per_w)],
            idx_v.at[j], sem_i)
    h.wait()

    def gather(u, buf):
        j, k = u // kchunks, u % kchunks
        return pltpu.async_copy(
            tab_hbm.at[idx_v.at[j, pl.ds(k * GIDX, GIDX)]],
            rows_v.at[buf], sem_g)

    def store(u, buf):
        j, k = u // kchunks, u % kchunks
        return pltpu.async_copy(
            rows_v.at[buf],
            cat_hbm.at[j, pl.ds(2 * (r0 + k * CROWS), GIDX)], sem_s)

    g = gather(0, 0)
    st = None
    for u in range(nunits):
        cur, nxt = u % 2, (u + 1) % 2
        g.wait()
        if u + 1 < nunits:
            if st is not None:
                st.wait()
            g = gather(u + 1, nxt)
        st = store(u, cur)
    st.wait()


def _sc_gather(tab, idxp, batch):
    return pl.kernel(
        _gather_body,
        out_type=jax.ShapeDtypeStruct((N_PAIR, 2 * batch, EMBED), jnp.bfloat16),
        mesh=plsc.VectorSubcoreMesh(
            core_axis_name="c", subcore_axis_name="s",
            num_cores=NC, num_subcores=NS),
        scratch_types=[
            pltpu.VMEM((N_PAIR, 2 * batch // NW), jnp.int32),
            pltpu.VMEM((2, GIDX, EMBED), jnp.bfloat16),
            pltpu.SemaphoreType.DMA,
            pltpu.SemaphoreType.DMA,
            pltpu.SemaphoreType.DMA,
        ],
        compiler_params=pltpu.CompilerParams(use_tc_tiling_on_sc=False),
    )(tab, idxp)


def _mm_body(cat_ref, w_ref, b_ref, o_ref):
    acc = b_ref[...].astype(jnp.float32)
    for j in range(N_PAIR):
        acc = acc + jnp.dot(cat_ref[j], w_ref[j],
                            preferred_element_type=jnp.float32)
    o_ref[...] = acc


def _tc_matmul(cat_p, w4, b2d):
    batch = cat_p.shape[1]
    bm = 1024
    return pl.pallas_call(
        _mm_body,
        grid=(batch // bm,),
        in_specs=[
            pl.BlockSpec((N_PAIR, bm, LANE), lambda i: (0, i, 0)),
            pl.BlockSpec((N_PAIR, LANE, OUT_DIM), lambda i: (0, 0, 0)),
            pl.BlockSpec((1, OUT_DIM), lambda i: (0, 0)),
        ],
        out_specs=pl.BlockSpec((bm, OUT_DIM), lambda i: (i, 0)),
        out_shape=jax.ShapeDtypeStruct((batch, OUT_DIM), jnp.float32),
    )(cat_p, w4, b2d)


def kernel(table0, table1, table2, table3, table4, table5, table6, table7,
           W, b, prev_action):
    tables = [table0, table1, table2, table3, table4, table5, table6, table7]
    batch = prev_action.shape[0]
    tab = jnp.concatenate(tables, axis=0).astype(jnp.bfloat16)
    cidx = (prev_action.astype(jnp.int32)
            + jnp.arange(N_CAT, dtype=jnp.int32) * VOCAB)
    idxp = cidx.reshape(batch, N_PAIR, 2).transpose(1, 0, 2).reshape(-1)
    w4 = W.reshape(N_PAIR, LANE, OUT_DIM).astype(jnp.bfloat16)
    cat_p = _sc_gather(tab, idxp, batch).reshape(N_PAIR, batch, LANE)
    return _tc_matmul(cat_p, w4, b.reshape(1, OUT_DIM))


# speedup vs baseline: 1.4617x; 1.4617x over previous
import functools

import jax
import jax.numpy as jnp
from jax import lax
from jax.experimental import pallas as pl
from jax.experimental.pallas import tpu as pltpu
from jax.experimental.pallas import tpu_sc as plsc

N_CAT = 8
N_PAIR = 4
VOCAB = 1000
EMBED = 64
OUT_DIM = 512
LANE = 2 * EMBED
SLICE_I32 = EMBED // 2

NC, NS = 2, 16
NW = NC * NS
GIDX = 128
HROWS = 32


def _gather_body(tab_hbm, idx_hbm, cat_hbm, idx_v, idxg_v, rows_v,
                 sem_i, sem_g, sem_s):
    batch = idx_hbm.shape[0] // N_CAT
    half = batch // 2
    hrows_per_w = half // NW
    kchunks = hrows_per_w // HROWS
    nunits = N_PAIR * kchunks
    wid = lax.axis_index("s") * NC + lax.axis_index("c")
    b0 = wid * hrows_per_w

    n_half = hrows_per_w * N_CAT
    pltpu.async_copy(idx_hbm.at[pl.ds(b0 * N_CAT, n_half)],
                     idx_v.at[pl.ds(0, n_half)], sem_i)
    h = pltpu.async_copy(idx_hbm.at[pl.ds((half + b0) * N_CAT, n_half)],
                         idx_v.at[pl.ds(n_half, n_half)], sem_i)
    h.wait()
    h.wait()

    e = lax.iota(jnp.int32, 16)
    pat = ((e >> 2) * N_CAT + (e & 1)
           + ((e >> 1) & 1) * (hrows_per_w * N_CAT))
    voff = (e & 1) * VOCAB
    for j in range(N_PAIR):
        offs = pat + 2 * j
        add = voff + 2 * j * VOCAB
        for k in range(kchunks):
            for p in range(8):
                src = plsc.load_gather(
                    idx_v, [offs + (HROWS * k + 4 * p) * N_CAT])
                idxg_v[j, pl.ds(k * GIDX + p * 16, 16)] = src + add

    def gather(u, buf):
        j, k = u // kchunks, u % kchunks
        return pltpu.async_copy(
            tab_hbm.at[idxg_v.at[j, pl.ds(k * GIDX, GIDX)]],
            rows_v.at[buf], sem_g)

    def store(u, buf):
        j, k = u // kchunks, u % kchunks
        return pltpu.async_copy(
            rows_v.at[buf],
            cat_hbm.at[j, pl.ds(4 * (b0 + HROWS * k), GIDX)], sem_s)

    g = gather(0, 0)
    st = None
    for u in range(nunits):
        cur, nxt = u % 2, (u + 1) % 2
        g.wait()
        if u + 1 < nunits:
            if st is not None:
                st.wait()
            g = gather(u + 1, nxt)
        st = store(u, cur)
    st.wait()


def _sc_gather(tab_i, idx, batch):
    return pl.kernel(
        _gather_body,
        out_type=jax.ShapeDtypeStruct((N_PAIR, 2 * batch, SLICE_I32),
                                      jnp.int32),
        mesh=plsc.VectorSubcoreMesh(
            core_axis_name="c", subcore_axis_name="s",
            num_cores=NC, num_subcores=NS),
        scratch_types=[
            pltpu.VMEM((batch // NW * N_CAT,), jnp.int32),
            pltpu.VMEM((N_PAIR, (batch // 2 // NW // HROWS) * GIDX),
                       jnp.int32),
            pltpu.VMEM((2, GIDX, SLICE_I32), jnp.int32),
            pltpu.SemaphoreType.DMA,
            pltpu.SemaphoreType.DMA,
            pltpu.SemaphoreType.DMA,
        ],
        compiler_params=pltpu.CompilerParams(use_tc_tiling_on_sc=False,
                                             needs_layout_passes=False),
    )(tab_i, idx)


def _mm_body(cat_ref, wev_ref, wod_ref, b_ref, o_ref):
    h = pl.program_id(1)
    acc = b_ref[...].astype(jnp.float32)
    for j in range(N_PAIR):
        a = cat_ref[j]
        ev = jax.lax.bitcast_convert_type(a << 16, jnp.float32)
        od = jax.lax.bitcast_convert_type(a & jnp.int32(-65536), jnp.float32)
        wev = jnp.where(h == 0, wev_ref[0, j], wev_ref[1, j])
        wod = jnp.where(h == 0, wod_ref[0, j], wod_ref[1, j])
        acc = acc + jnp.dot(ev.astype(jnp.bfloat16), wev,
                            preferred_element_type=jnp.float32)
        acc = acc + jnp.dot(od.astype(jnp.bfloat16), wod,
                            preferred_element_type=jnp.float32)
    o_ref[...] = acc


def _tc_matmul(cat_i, w_ev, w_od, b2d, batch):
    bm = 1024
    nblk = batch // 2 // bm
    k2 = 2 * EMBED
    return pl.pallas_call(
        _mm_body,
        grid=(nblk, 2),
        in_specs=[
            pl.BlockSpec((N_PAIR, bm, k2), lambda i, h: (0, i, 0)),
            pl.BlockSpec((2, N_PAIR, k2, OUT_DIM), lambda i, h: (0, 0, 0, 0)),
            pl.BlockSpec((2, N_PAIR, k2, OUT_DIM), lambda i, h: (0, 0, 0, 0)),
            pl.BlockSpec((1, OUT_DIM), lambda i, h: (0, 0)),
        ],
        out_specs=pl.BlockSpec((bm, OUT_DIM), lambda i, h: (i + nblk * h, 0)),
        out_shape=jax.ShapeDtypeStruct((batch, OUT_DIM), jnp.float32),
    )(cat_i, w_ev, w_od, b2d)


def kernel(table0, table1, table2, table3, table4, table5, table6, table7,
           W, b, prev_action):
    tables = [table0, table1, table2, table3, table4, table5, table6, table7]
    batch = prev_action.shape[0]
    tab_bf = jnp.concatenate(tables, axis=0).astype(jnp.bfloat16)
    tab_i = jax.lax.bitcast_convert_type(
        tab_bf.reshape(N_CAT * VOCAB, SLICE_I32, 2), jnp.int32)
    idx = prev_action.astype(jnp.int32).reshape(-1)
    w5 = W.reshape(N_PAIR, EMBED, 2, OUT_DIM)
    wev1, wod1 = w5[:, :, 0, :], w5[:, :, 1, :]
    zed = jnp.zeros_like(wev1)
    w_ev = jnp.stack([jnp.concatenate([wev1, zed], axis=1),
                      jnp.concatenate([zed, wev1], axis=1)]
                     ).astype(jnp.bfloat16)
    w_od = jnp.stack([jnp.concatenate([wod1, zed], axis=1),
                      jnp.concatenate([zed, wod1], axis=1)]
                     ).astype(jnp.bfloat16)
    cat_i = _sc_gather(tab_i, idx, batch)
    cat_i = cat_i.reshape(N_PAIR, batch // 2, 4 * SLICE_I32)
    return _tc_matmul(cat_i, w_ev, w_od, b.reshape(1, OUT_DIM), batch)


# speedup vs baseline: 1.5989x; 1.0939x over previous
import functools

import jax
import jax.numpy as jnp
from jax import lax
from jax.experimental import pallas as pl
from jax.experimental.pallas import tpu as pltpu
from jax.experimental.pallas import tpu_sc as plsc

N_CAT = 8
N_PAIR = 4
VOCAB = 1000
EMBED = 64
OUT_DIM = 512
LANE = 2 * EMBED
SLICE_I32 = EMBED // 2

NC, NS = 2, 16
NW = NC * NS
GIDX = 128
HROWS = 32


def _gather_body(tab_hbm, idx_hbm, cat_hbm, idx_v, idxg_v, rows_v,
                 sem_i, sem_g, sem_s):
    batch = idx_hbm.shape[0]
    half = batch // 2
    hrows_per_w = half // NW
    kchunks = hrows_per_w // HROWS
    nunits = N_PAIR * kchunks
    wid = lax.axis_index("s") * NC + lax.axis_index("c")
    b0 = wid * hrows_per_w

    pltpu.async_copy(idx_hbm.at[pl.ds(b0, hrows_per_w)],
                     idx_v.at[pl.ds(0, hrows_per_w)], sem_i)
    h = pltpu.async_copy(idx_hbm.at[pl.ds(half + b0, hrows_per_w)],
                         idx_v.at[pl.ds(hrows_per_w, hrows_per_w)], sem_i)
    h.wait()
    h.wait()

    e = lax.iota(jnp.int32, 16)
    rowpat = (e >> 2) + ((e >> 1) & 1) * hrows_per_w
    colpat = e & 1
    voff = (e & 1) * VOCAB
    for j in range(N_PAIR):
        add = voff + 2 * j * VOCAB
        for k in range(kchunks):
            for p in range(8):
                src = plsc.load_gather(
                    idx_v, [rowpat + (HROWS * k + 4 * p), colpat + 2 * j])
                idxg_v[j, pl.ds(k * GIDX + p * 16, 16)] = src + add

    def gather(u, buf):
        j, k = u // kchunks, u % kchunks
        return pltpu.async_copy(
            tab_hbm.at[idxg_v.at[j, pl.ds(k * GIDX, GIDX)]],
            rows_v.at[buf], sem_g)

    def store(u, buf):
        j, k = u // kchunks, u % kchunks
        return pltpu.async_copy(
            rows_v.at[buf],
            cat_hbm.at[j, pl.ds(4 * (b0 + HROWS * k), GIDX)], sem_s)

    NBUF, DEPTH = 8, 4
    g = [None] * nunits
    st = [None] * nunits

    def fire(v):
        if v >= NBUF:
            st[v - NBUF].wait()
        g[v] = gather(v, v % NBUF)

    for v in range(DEPTH):
        fire(v)
    for u in range(nunits):
        g[u].wait()
        st[u] = store(u, u % NBUF)
        if u + DEPTH < nunits:
            fire(u + DEPTH)
    for v in range(nunits - NBUF, nunits):
        st[v].wait()


def _sc_gather(tab_i, idx, batch):
    return pl.kernel(
        _gather_body,
        out_type=jax.ShapeDtypeStruct((N_PAIR, 2 * batch, SLICE_I32),
                                      jnp.int32),
        mesh=plsc.VectorSubcoreMesh(
            core_axis_name="c", subcore_axis_name="s",
            num_cores=NC, num_subcores=NS),
        scratch_types=[
            pltpu.VMEM((batch // NW, N_CAT), jnp.int32),
            pltpu.VMEM((N_PAIR, (batch // 2 // NW // HROWS) * GIDX),
                       jnp.int32),
            pltpu.VMEM((8, GIDX, SLICE_I32), jnp.int32),
            pltpu.SemaphoreType.DMA,
            pltpu.SemaphoreType.DMA,
            pltpu.SemaphoreType.DMA,
        ],
        compiler_params=pltpu.CompilerParams(use_tc_tiling_on_sc=False,
                                             needs_layout_passes=False),
    )(tab_i, idx)


def _mm_body(cat_ref, wev_ref, wod_ref, b_ref, o_ref):
    h = pl.program_id(1)
    acc = b_ref[...].astype(jnp.float32)
    for j in range(N_PAIR):
        a = cat_ref[j]
        ev = jax.lax.bitcast_convert_type(a << 16, jnp.float32)
        od = jax.lax.bitcast_convert_type(a & jnp.int32(-65536), jnp.float32)
        wev = jnp.where(h == 0, wev_ref[0, j], wev_ref[1, j])
        wod = jnp.where(h == 0, wod_ref[0, j], wod_ref[1, j])
        acc = acc + jnp.dot(ev.astype(jnp.bfloat16), wev,
                            preferred_element_type=jnp.float32)
        acc = acc + jnp.dot(od.astype(jnp.bfloat16), wod,
                            preferred_element_type=jnp.float32)
    o_ref[...] = acc


def _tc_matmul(cat_i, w_ev, w_od, b2d, batch):
    bm = 1024
    nblk = batch // 2 // bm
    k2 = 2 * EMBED
    return pl.pallas_call(
        _mm_body,
        grid=(nblk, 2),
        in_specs=[
            pl.BlockSpec((N_PAIR, bm, k2), lambda i, h: (0, i, 0)),
            pl.BlockSpec((2, N_PAIR, k2, OUT_DIM), lambda i, h: (0, 0, 0, 0)),
            pl.BlockSpec((2, N_PAIR, k2, OUT_DIM), lambda i, h: (0, 0, 0, 0)),
            pl.BlockSpec((1, OUT_DIM), lambda i, h: (0, 0)),
        ],
        out_specs=pl.BlockSpec((bm, OUT_DIM), lambda i, h: (i + nblk * h, 0)),
        out_shape=jax.ShapeDtypeStruct((batch, OUT_DIM), jnp.float32),
    )(cat_i, w_ev, w_od, b2d)


def kernel(table0, table1, table2, table3, table4, table5, table6, table7,
           W, b, prev_action):
    tables = [table0, table1, table2, table3, table4, table5, table6, table7]
    batch = prev_action.shape[0]
    tab_bf = jnp.concatenate(tables, axis=0).astype(jnp.bfloat16)
    tab_i = jax.lax.bitcast_convert_type(
        tab_bf.reshape(N_CAT * VOCAB, SLICE_I32, 2), jnp.int32)
    idx = prev_action.astype(jnp.int32)
    w5 = W.reshape(N_PAIR, EMBED, 2, OUT_DIM)
    wev1, wod1 = w5[:, :, 0, :], w5[:, :, 1, :]
    zed = jnp.zeros_like(wev1)
    w_ev = jnp.stack([jnp.concatenate([wev1, zed], axis=1),
                      jnp.concatenate([zed, wev1], axis=1)]
                     ).astype(jnp.bfloat16)
    w_od = jnp.stack([jnp.concatenate([wod1, zed], axis=1),
                      jnp.concatenate([zed, wod1], axis=1)]
                     ).astype(jnp.bfloat16)
    cat_i = _sc_gather(tab_i, idx, batch)
    cat_i = cat_i.reshape(N_PAIR, batch // 2, 4 * SLICE_I32)
    return _tc_matmul(cat_i, w_ev, w_od, b.reshape(1, OUT_DIM), batch)
